# baseline (speedup 1.0000x reference)
import functools

import jax
import jax.numpy as jnp
from jax import lax
from jax.experimental import pallas as pl
from jax.experimental.pallas import tpu as pltpu
from jax.experimental.pallas import tpu_sc as plsc

L, B, V, E, T = 200, 4096, 1000000, 32, 16
N = L * B
NW = 32
ROWS_PER_W = N // NW
CHUNK = 512
G = 128
NGATHER = CHUNK // G
NCHUNK = ROWS_PER_W // CHUNK
GROUPS = CHUNK // 16


def _body(t_hbm, m_hbm, len_hbm, table_hbm, w_hbm, b_hbm, out_hbm,
          idx_v, t_v, m_buf, out48, w_v, b_v, len_v,
          isem, gsem, tsem, osem):
    wid = lax.axis_index("s") * 2 + lax.axis_index("c")
    wbase = wid * ROWS_PER_W

    pltpu.sync_copy(w_hbm, w_v)
    pltpu.sync_copy(b_hbm, b_v)
    pltpu.sync_copy(len_hbm, len_v)
    wy = w_v[pl.ds(0, 16)]
    wm = w_v[pl.ds(16, 16)]
    wd = w_v[pl.ds(32, 16)]
    bv = b_v[...]

    def fire_in(c, b):
        base = wbase + c * CHUNK
        for j in range(NGATHER):
            pltpu.async_copy(
                table_hbm.at[idx_v[b].at[pl.ds(j * G, G)]],
                m_buf[b].at[pl.ds(j * G, G)], gsem)
        pltpu.async_copy(t_hbm.at[pl.ds(base, CHUNK)], t_v[b], tsem)

    def wait_in(c, b):
        base = wbase + c * CHUNK
        for j in range(NGATHER):
            pltpu.make_async_copy(
                table_hbm.at[idx_v[b].at[pl.ds(j * G, G)]],
                m_buf[b].at[pl.ds(j * G, G)], gsem).wait()
        pltpu.make_async_copy(t_hbm.at[pl.ds(base, CHUNK)], t_v[b], tsem).wait()

    def fire_idx(c, b):
        pltpu.async_copy(
            m_hbm.at[pl.ds(wbase + c * CHUNK, CHUNK)], idx_v[b], isem)

    def wait_idx(c, b):
        pltpu.make_async_copy(
            m_hbm.at[pl.ds(wbase + c * CHUNK, CHUNK)], idx_v[b], isem).wait()

    def out_dma(c, b):
        return pltpu.make_async_copy(
            out48[b], out_hbm.at[pl.ds(wbase + c * CHUNK, CHUNK)], osem)

    def compute(c, b):
        li = (wbase + c * CHUNK) // B
        b0 = (wbase + c * CHUNK) % B

        def group_body(g, carry):
            r0 = g * 16
            days = t_v[b][pl.ds(r0, 16)]
            idxv = idx_v[b][pl.ds(r0, 16)]
            lenb = len_v[pl.ds(b0 + r0, 16)]
            padv = jnp.where(li < lenb, 1.0, 0.0)
            years = (days / 365.0).astype(jnp.int32).astype(jnp.float32)
            rem = days - years * 365.0
            months = (rem / 30.0).astype(jnp.int32).astype(jnp.float32)
            dd = rem - months * 30.0
            years = years * padv
            months = months * padv
            dd = dd * padv
            keep = jnp.where((idxv != 0) & (li < lenb), 1.0, 0.0)
            for i in range(16):
                r = r0 + i
                trow = years[i] * wy + months[i] * wm + dd[i] * wd + padv[i] * bv
                kb = keep[i]
                out48[b][r, pl.ds(0, 16)] = trow
                out48[b][r, pl.ds(16, 16)] = m_buf[b][r, pl.ds(0, 16)] * kb
                out48[b][r, pl.ds(32, 16)] = m_buf[b][r, pl.ds(16, 16)] * kb
            return carry

        lax.fori_loop(0, GROUPS, group_body, 0, unroll=False)

    fire_idx(0, 0)
    wait_idx(0, 0)
    fire_in(0, 0)
    fire_idx(1, 1)

    def pair_body(p, carry):
        for b in range(2):
            c = p * 2 + b
            @pl.when(c >= 2)
            def _():
                out_dma(c - 2, b).wait()

            wait_in(c, b)

            @pl.when(c + 1 < NCHUNK)
            def _():
                wait_idx(c + 1, 1 - b)
                fire_in(c + 1, 1 - b)

            @pl.when(c + 2 < NCHUNK)
            def _():
                fire_idx(c + 2, b)

            compute(c, b)
            out_dma(c, b).start()
        return carry

    lax.fori_loop(0, NCHUNK // 2, pair_body, 0, unroll=False)

    out_dma(NCHUNK - 2, 0).wait()
    out_dma(NCHUNK - 1, 1).wait()


@jax.jit
def _run(days, m1f, lenf, emb_table, W_t, b_t):
    mesh = plsc.VectorSubcoreMesh(core_axis_name="c", subcore_axis_name="s")
    kfn = functools.partial(
        pl.kernel,
        mesh=mesh,
        compiler_params=pltpu.CompilerParams(
            needs_layout_passes=False, use_tc_tiling_on_sc=False),
        out_type=jax.ShapeDtypeStruct((N, T + E), jnp.float32),
        scratch_types=[
            [pltpu.VMEM((CHUNK,), jnp.int32) for _ in range(2)],
            [pltpu.VMEM((CHUNK,), jnp.float32) for _ in range(2)],
            [pltpu.VMEM((CHUNK, E), jnp.float32) for _ in range(2)],
            [pltpu.VMEM((CHUNK, T + E), jnp.float32) for _ in range(2)],
            pltpu.VMEM((3 * T,), jnp.float32),
            pltpu.VMEM((T,), jnp.float32),
            pltpu.VMEM((B,), jnp.int32),
            pltpu.SemaphoreType.DMA,
            pltpu.SemaphoreType.DMA,
            pltpu.SemaphoreType.DMA,
            pltpu.SemaphoreType.DMA,
        ],
    )(_body)
    return kfn(days, m1f, lenf, emb_table, W_t, b_t)


def kernel(t1, m1, len1, pad_mask, emb_table, W_t, b_t):
    days = t1.reshape(N) / 86400.0
    m1f = m1.reshape(N).astype(jnp.int32)
    lenf = len1.reshape(B).astype(jnp.int32)
    out = _run(days, m1f, lenf, emb_table, W_t.T.reshape(3 * T), b_t)
    return out.reshape(L, B, T + E)


# speedup vs baseline: 1.0009x; 1.0009x over previous
import functools

import jax
import jax.numpy as jnp
from jax import lax
from jax.experimental import pallas as pl
from jax.experimental.pallas import tpu as pltpu
from jax.experimental.pallas import tpu_sc as plsc

L, B, V, E, T = 200, 4096, 1000000, 32, 16
N = L * B
NW = 32
ROWS_PER_W = N // NW
CHUNK = 512
G = 128
NGATHER = CHUNK // G
NCHUNK = ROWS_PER_W // CHUNK
GROUPS = CHUNK // 16


def _body(t_hbm, m_hbm, len_hbm, table_hbm, w_hbm, b_hbm, out_hbm,
          idx_v, t_v, m_buf, out48, w_v, b_v, len_v,
          isem, gsem, tsem, osem):
    wid = lax.axis_index("s") * 2 + lax.axis_index("c")
    wbase = wid * ROWS_PER_W

    pltpu.sync_copy(w_hbm, w_v)
    pltpu.sync_copy(b_hbm, b_v)
    pltpu.sync_copy(len_hbm, len_v)
    wy = w_v[pl.ds(0, 16)]
    wm = w_v[pl.ds(16, 16)]
    wd = w_v[pl.ds(32, 16)]
    bv = b_v[...]

    def fire_in(c, b):
        base = wbase + c * CHUNK
        for j in range(NGATHER):
            pltpu.async_copy(
                table_hbm.at[idx_v[b].at[pl.ds(j * G, G)]],
                m_buf[b].at[pl.ds(j * G, G)], gsem)
        pltpu.async_copy(t_hbm.at[pl.ds(base, CHUNK)], t_v[b], tsem)

    def wait_in(c, b):
        base = wbase + c * CHUNK
        for j in range(NGATHER):
            pltpu.make_async_copy(
                table_hbm.at[idx_v[b].at[pl.ds(j * G, G)]],
                m_buf[b].at[pl.ds(j * G, G)], gsem).wait()
        pltpu.make_async_copy(t_hbm.at[pl.ds(base, CHUNK)], t_v[b], tsem).wait()

    def fire_idx(c, b):
        pltpu.async_copy(
            m_hbm.at[pl.ds(wbase + c * CHUNK, CHUNK)], idx_v[b], isem)

    def wait_idx(c, b):
        pltpu.make_async_copy(
            m_hbm.at[pl.ds(wbase + c * CHUNK, CHUNK)], idx_v[b], isem).wait()

    def out_dma(c, b):
        base = wbase + c * CHUNK
        return pltpu.make_async_copy(
            out48[b], out_hbm.at[base // B, pl.ds(base % B, CHUNK)], osem)

    def compute(c, b):
        li = (wbase + c * CHUNK) // B
        b0 = (wbase + c * CHUNK) % B

        def group_body(g, carry):
            r0 = g * 16
            days = t_v[b][pl.ds(r0, 16)]
            idxv = idx_v[b][pl.ds(r0, 16)]
            lenb = len_v[pl.ds(b0 + r0, 16)]
            padv = jnp.where(li < lenb, 1.0, 0.0)
            years = (days / 365.0).astype(jnp.int32).astype(jnp.float32)
            rem = days - years * 365.0
            months = (rem / 30.0).astype(jnp.int32).astype(jnp.float32)
            dd = rem - months * 30.0
            years = years * padv
            months = months * padv
            dd = dd * padv
            keep = jnp.where((idxv != 0) & (li < lenb), 1.0, 0.0)
            for i in range(16):
                r = r0 + i
                trow = years[i] * wy + months[i] * wm + dd[i] * wd + padv[i] * bv
                kb = keep[i]
                out48[b][r, pl.ds(0, 16)] = trow
                out48[b][r, pl.ds(16, 16)] = m_buf[b][r, pl.ds(0, 16)] * kb
                out48[b][r, pl.ds(32, 16)] = m_buf[b][r, pl.ds(16, 16)] * kb
            return carry

        lax.fori_loop(0, GROUPS, group_body, 0, unroll=False)

    fire_idx(0, 0)
    wait_idx(0, 0)
    fire_in(0, 0)
    fire_idx(1, 1)

    def pair_body(p, carry):
        for b in range(2):
            c = p * 2 + b
            @pl.when(c >= 2)
            def _():
                out_dma(c - 2, b).wait()

            wait_in(c, b)

            @pl.when(c + 1 < NCHUNK)
            def _():
                wait_idx(c + 1, 1 - b)
                fire_in(c + 1, 1 - b)

            @pl.when(c + 2 < NCHUNK)
            def _():
                fire_idx(c + 2, b)

            compute(c, b)
            out_dma(c, b).start()
        return carry

    lax.fori_loop(0, NCHUNK // 2, pair_body, 0, unroll=False)

    out_dma(NCHUNK - 2, 0).wait()
    out_dma(NCHUNK - 1, 1).wait()


@jax.jit
def _run(days, m1f, lenf, emb_table, W_t, b_t):
    mesh = plsc.VectorSubcoreMesh(core_axis_name="c", subcore_axis_name="s")
    kfn = functools.partial(
        pl.kernel,
        mesh=mesh,
        compiler_params=pltpu.CompilerParams(
            needs_layout_passes=False, use_tc_tiling_on_sc=False),
        out_type=jax.ShapeDtypeStruct((L, B, T + E), jnp.float32),
        scratch_types=[
            [pltpu.VMEM((CHUNK,), jnp.int32) for _ in range(2)],
            [pltpu.VMEM((CHUNK,), jnp.float32) for _ in range(2)],
            [pltpu.VMEM((CHUNK, E), jnp.float32) for _ in range(2)],
            [pltpu.VMEM((CHUNK, T + E), jnp.float32) for _ in range(2)],
            pltpu.VMEM((3 * T,), jnp.float32),
            pltpu.VMEM((T,), jnp.float32),
            pltpu.VMEM((B,), jnp.int32),
            pltpu.SemaphoreType.DMA,
            pltpu.SemaphoreType.DMA,
            pltpu.SemaphoreType.DMA,
            pltpu.SemaphoreType.DMA,
        ],
    )(_body)
    return kfn(days, m1f, lenf, emb_table, W_t, b_t)


def kernel(t1, m1, len1, pad_mask, emb_table, W_t, b_t):
    days = t1.reshape(N) / 86400.0
    m1f = m1.reshape(N).astype(jnp.int32)
    lenf = len1.reshape(B).astype(jnp.int32)
    return _run(days, m1f, lenf, emb_table, W_t.T.reshape(3 * T), b_t)
